# Initial kernel scaffold; baseline (speedup 1.0000x reference)
import jax
import jax.numpy as jnp
from jax.experimental import pallas as pl


def kernel(x, Wq, bq, Wk, bk, Wv, bv, Wo, bo, log_tau, lambda_auto, beta_lag, log_tau_lag):
    raise NotImplementedError("write your pallas kernel here")


# speedup vs baseline: 13.2716x; 13.2716x over previous
import functools
import math

import jax
import jax.numpy as jnp
from jax import lax
from jax.experimental import pallas as pl
from jax.experimental.pallas import tpu as pltpu

_B, _T, _D, _H = 1, 2048, 1024, 16
_Dh = _D // _H
_MAX_LAG = 32
_C_TOPK = 1


def _lag_candidates(t, max_lag):
    if t <= 1:
        return []
    if max_lag is None or max_lag >= t - 1:
        return list(range(1, t))
    step = max(1, (t - 1) // max_lag)
    cand = list(range(1, t, step))[:max_lag]
    if cand and cand[-1] != t - 1:
        cand[-1] = t - 1
    return cand


_LAGS = _lag_candidates(_T, _MAX_LAG)
_L = len(_LAGS)
_K_TOP = max(1, int(_C_TOPK * math.ceil(math.log(max(_T, 2), 2))))
_K_EFF = min(_K_TOP, _L)


def _tdot(a, b):
    return lax.dot_general(a, b, (((0,), (0,)), ((), ())),
                           preferred_element_type=jnp.float32)


def _matmul_bias_kernel(x_ref, w_ref, b_ref, o_ref):
    o_ref[:] = jnp.dot(x_ref[:], w_ref[:],
                       preferred_element_type=jnp.float32) + b_ref[:]


def _matmul_bias(x, w, b, tile_m=256, tile_n=1024):
    m, k = x.shape
    _, n = w.shape
    return pl.pallas_call(
        _matmul_bias_kernel,
        grid=(m // tile_m, n // tile_n),
        in_specs=[
            pl.BlockSpec((tile_m, k), lambda i, j: (i, 0)),
            pl.BlockSpec((k, tile_n), lambda i, j: (0, j)),
            pl.BlockSpec((1, tile_n), lambda i, j: (0, j)),
        ],
        out_specs=pl.BlockSpec((tile_m, tile_n), lambda i, j: (i, j)),
        out_shape=jax.ShapeDtypeStruct((m, n), jnp.float32),
    )(x, w, b)


def _softmax_rows(z):
    z = z - jnp.max(z, axis=-1, keepdims=True)
    e = jnp.exp(z)
    return e / jnp.sum(e, axis=-1, keepdims=True)


def _head_kernel(scal_ref, q_ref, k_ref, v_ref, o_ref, qext_ref, covs_ref,
                 yext_ref):
    tau = scal_ref[0, 0]
    lam = scal_ref[0, 1]
    beta = scal_ref[0, 2]
    tau_lag = scal_ref[0, 3]

    q = q_ref[0]
    k = k_ref[0]
    v = v_ref[0]
    qn = q / jnp.sqrt(jnp.maximum(jnp.sum(q * q, axis=0, keepdims=True), 1e-8))
    kn = k / jnp.sqrt(jnp.maximum(jnp.sum(k * k, axis=0, keepdims=True), 1e-8))

    qext_ref[0:_T, :] = qn
    qext_ref[_T:2 * _T, :] = qn

    cov0 = _tdot(kn, qn)
    att0 = _softmax_rows(cov0 / tau)
    inst = jnp.dot(v, att0, preferred_element_type=jnp.float32)

    eye = (lax.broadcasted_iota(jnp.int32, (_Dh, _Dh), 0) ==
           lax.broadcasted_iota(jnp.int32, (_Dh, _Dh), 1))
    scores = []
    for l, lag in enumerate(_LAGS):
        qw = qext_ref[lag:lag + _T, :]
        covl = _tdot(kn, qw)
        covs_ref[l * _Dh:(l + 1) * _Dh, :] = covl
        a = jnp.abs(covl)
        dsum = jnp.sum(jnp.where(eye, a, 0.0))
        tsum = jnp.sum(a)
        scores.append((lam * dsum + (1.0 - lam) * (tsum - dsum)).reshape(1, 1))
    scores = jnp.concatenate(scores, axis=1)

    iota_l = lax.broadcasted_iota(jnp.int32, (1, _L), 1)

    cur = scores
    sel_idx, sel_sc = [], []
    for _ in range(_K_EFF):
        m = jnp.max(cur)
        idx = jnp.min(jnp.where(cur == m, iota_l, _L))
        sel_idx.append(idx)
        sel_sc.append(m)
        cur = jnp.where(iota_l == idx, -jnp.inf, cur)

    mmax = functools.reduce(jnp.maximum, sel_sc)
    es = [jnp.exp((s - mmax) / tau_lag) for s in sel_sc]
    denom = functools.reduce(lax.add, es)

    acc = (1.0 - beta) * inst
    for kk in range(_K_EFF):
        idx = sel_idx[kk]
        covk = covs_ref[pl.ds(idx * _Dh, _Dh), :]
        attk = _softmax_rows(covk / tau)
        wk = beta * es[kk] / denom
        y = jnp.dot(v, attk * wk, preferred_element_type=jnp.float32)
        yext_ref[0:_T, :] = y
        yext_ref[_T:2 * _T, :] = y
        lagk = functools.reduce(lax.add, [
            jnp.where(idx == l, lag, 0) for l, lag in enumerate(_LAGS)])
        acc = acc + yext_ref[pl.ds(_T - lagk, _T), :]
    o_ref[0] = acc


def _head_stage(qkv3, scal):
    return pl.pallas_call(
        _head_kernel,
        grid=(_H,),
        in_specs=[
            pl.BlockSpec(memory_space=pltpu.SMEM),
            pl.BlockSpec((1, _T, _Dh), lambda h: (h, 0, 0)),
            pl.BlockSpec((1, _T, _Dh), lambda h: (_H + h, 0, 0)),
            pl.BlockSpec((1, _T, _Dh), lambda h: (2 * _H + h, 0, 0)),
        ],
        out_specs=pl.BlockSpec((1, _T, _Dh), lambda h: (h, 0, 0)),
        out_shape=jax.ShapeDtypeStruct((_H, _T, _Dh), jnp.float32),
        scratch_shapes=[
            pltpu.VMEM((2 * _T, _Dh), jnp.float32),
            pltpu.VMEM((_L * _Dh, _Dh), jnp.float32),
            pltpu.VMEM((2 * _T, _Dh), jnp.float32),
        ],
    )(scal, qkv3, qkv3, qkv3)


def kernel(x, Wq, bq, Wk, bk, Wv, bv, Wo, bo, log_tau, lambda_auto, beta_lag,
           log_tau_lag):
    x2 = x.reshape(_T, _D)
    w_all = jnp.concatenate([Wq.T, Wk.T, Wv.T], axis=1)
    b_all = jnp.concatenate([bq, bk, bv]).reshape(1, 3 * _D)

    qkv = _matmul_bias(x2, w_all, b_all)
    qkv3 = qkv.reshape(_T, 3 * _H, _Dh).transpose(1, 0, 2)

    tau = jnp.maximum(jnp.exp(log_tau[0]), 1e-4)
    tau_lag = jnp.maximum(jnp.exp(log_tau_lag[0]), 1e-4)
    lam = jnp.clip(lambda_auto, 0.0, 1.0)
    beta = jnp.clip(beta_lag, 0.0, 1.0)
    scal = jnp.stack([tau, lam, beta, tau_lag]).reshape(1, 4)

    out_h = _head_stage(qkv3, scal)
    out_h2 = out_h.transpose(1, 0, 2).reshape(_T, _D)
    out = _matmul_bias(out_h2, Wo.T, bo.reshape(1, _D))
    return out.reshape(_B, _T, _D)


# speedup vs baseline: 16.3761x; 1.2339x over previous
import functools
import math

import jax
import jax.numpy as jnp
from jax import lax
from jax.experimental import pallas as pl
from jax.experimental.pallas import tpu as pltpu

_B, _T, _D, _H = 1, 2048, 1024, 16
_Dh = _D // _H
_MAX_LAG = 32
_C_TOPK = 1


def _lag_candidates(t, max_lag):
    if t <= 1:
        return []
    if max_lag is None or max_lag >= t - 1:
        return list(range(1, t))
    step = max(1, (t - 1) // max_lag)
    cand = list(range(1, t, step))[:max_lag]
    if cand and cand[-1] != t - 1:
        cand[-1] = t - 1
    return cand


_LAGS = _lag_candidates(_T, _MAX_LAG)
_L = len(_LAGS)
_K_TOP = max(1, int(_C_TOPK * math.ceil(math.log(max(_T, 2), 2))))
_K_EFF = min(_K_TOP, _L)


def _tdot(a, b):
    return lax.dot_general(a, b, (((0,), (0,)), ((), ())),
                           preferred_element_type=jnp.float32)


def _softmax_rows(z):
    z = z - jnp.max(z, axis=-1, keepdims=True)
    e = jnp.exp(z)
    return e / jnp.sum(e, axis=-1, keepdims=True)


def _head_body(q, k, v, tau, lam, beta, tau_lag, qext_ref, covs_ref, yext_ref):
    qn = q / jnp.sqrt(jnp.maximum(jnp.sum(q * q, axis=0, keepdims=True), 1e-8))
    kn = k / jnp.sqrt(jnp.maximum(jnp.sum(k * k, axis=0, keepdims=True), 1e-8))

    qext_ref[0:_T, :] = qn
    qext_ref[_T:2 * _T, :] = qn

    cov0 = _tdot(kn, qn)
    att0 = _softmax_rows(cov0 / tau)
    inst = jnp.dot(v, att0, preferred_element_type=jnp.float32)

    eye = (lax.broadcasted_iota(jnp.int32, (_Dh, _Dh), 0) ==
           lax.broadcasted_iota(jnp.int32, (_Dh, _Dh), 1))
    scores = []
    for l, lag in enumerate(_LAGS):
        qw = qext_ref[lag:lag + _T, :]
        covl = _tdot(kn, qw)
        covs_ref[l * _Dh:(l + 1) * _Dh, :] = covl
        a = jnp.abs(covl)
        dsum = jnp.sum(jnp.where(eye, a, 0.0))
        tsum = jnp.sum(a)
        scores.append((lam * dsum + (1.0 - lam) * (tsum - dsum)).reshape(1, 1))
    scores = jnp.concatenate(scores, axis=1)

    iota_l = lax.broadcasted_iota(jnp.int32, (1, _L), 1)
    cur = scores
    sel_idx, sel_sc = [], []
    for _ in range(_K_EFF):
        m = jnp.max(cur)
        idx = jnp.min(jnp.where(cur == m, iota_l, _L))
        sel_idx.append(idx)
        sel_sc.append(m)
        cur = jnp.where(iota_l == idx, -jnp.inf, cur)

    mmax = functools.reduce(jnp.maximum, sel_sc)
    es = [jnp.exp((s - mmax) / tau_lag) for s in sel_sc]
    denom = functools.reduce(lax.add, es)

    acc = (1.0 - beta) * inst
    for kk in range(_K_EFF):
        idx = sel_idx[kk]
        covk = covs_ref[pl.ds(idx * _Dh, _Dh), :]
        attk = _softmax_rows(covk / tau)
        wk = beta * es[kk] / denom
        y = jnp.dot(v, attk * wk, preferred_element_type=jnp.float32)
        yext_ref[0:_T, :] = y
        yext_ref[_T:2 * _T, :] = y
        lagk = functools.reduce(lax.add, [
            jnp.where(idx == l, lag, 0) for l, lag in enumerate(_LAGS)])
        acc = acc + yext_ref[pl.ds(_T - lagk, _T), :]
    return acc


def _fused_kernel(scal_ref, x_ref, wq_ref, wk_ref, wv_ref, bq_ref, bk_ref,
                  bv_ref, wo_ref, bo_ref, o_ref, qext_ref, covs_ref, yext_ref):
    g = pl.program_id(0)
    tau = scal_ref[0, 0]
    lam = scal_ref[0, 1]
    beta = scal_ref[0, 2]
    tau_lag = scal_ref[0, 3]

    xv = x_ref[:]
    qp = jnp.dot(xv, wq_ref[:], preferred_element_type=jnp.float32) + bq_ref[:]
    kp = jnp.dot(xv, wk_ref[:], preferred_element_type=jnp.float32) + bk_ref[:]
    vp = jnp.dot(xv, wv_ref[:], preferred_element_type=jnp.float32) + bv_ref[:]

    outs = []
    for s in range(2):
        sl = slice(s * _Dh, (s + 1) * _Dh)
        outs.append(_head_body(qp[:, sl], kp[:, sl], vp[:, sl], tau, lam,
                               beta, tau_lag, qext_ref, covs_ref, yext_ref))
    pair = jnp.concatenate(outs, axis=1)
    contrib = jnp.dot(pair, wo_ref[:], preferred_element_type=jnp.float32)

    @pl.when(g == 0)
    def _():
        o_ref[:] = contrib + bo_ref[:]

    @pl.when(g > 0)
    def _():
        o_ref[:] = o_ref[:] + contrib


def kernel(x, Wq, bq, Wk, bk, Wv, bv, Wo, bo, log_tau, lambda_auto, beta_lag,
           log_tau_lag):
    x2 = x.reshape(_T, _D)
    tau = jnp.maximum(jnp.exp(log_tau[0]), 1e-4)
    tau_lag = jnp.maximum(jnp.exp(log_tau_lag[0]), 1e-4)
    lam = jnp.clip(lambda_auto, 0.0, 1.0)
    beta = jnp.clip(beta_lag, 0.0, 1.0)
    scal = jnp.stack([tau, lam, beta, tau_lag]).reshape(1, 4)

    npair = _H // 2
    cw = 2 * _Dh
    out = pl.pallas_call(
        _fused_kernel,
        grid=(npair,),
        in_specs=[
            pl.BlockSpec(memory_space=pltpu.SMEM),
            pl.BlockSpec((_T, _D), lambda g: (0, 0)),
            pl.BlockSpec((_D, cw), lambda g: (0, g)),
            pl.BlockSpec((_D, cw), lambda g: (0, g)),
            pl.BlockSpec((_D, cw), lambda g: (0, g)),
            pl.BlockSpec((1, cw), lambda g: (0, g)),
            pl.BlockSpec((1, cw), lambda g: (0, g)),
            pl.BlockSpec((1, cw), lambda g: (0, g)),
            pl.BlockSpec((cw, _D), lambda g: (g, 0)),
            pl.BlockSpec((1, _D), lambda g: (0, 0)),
        ],
        out_specs=pl.BlockSpec((_T, _D), lambda g: (0, 0)),
        out_shape=jax.ShapeDtypeStruct((_T, _D), jnp.float32),
        scratch_shapes=[
            pltpu.VMEM((2 * _T, _Dh), jnp.float32),
            pltpu.VMEM((_L * _Dh, _Dh), jnp.float32),
            pltpu.VMEM((2 * _T, _Dh), jnp.float32),
        ],
    )(scal, x2, Wq.T, Wk.T, Wv.T, bq.reshape(1, _D), bk.reshape(1, _D),
      bv.reshape(1, _D), Wo.T, bo.reshape(1, _D))
    return out.reshape(_B, _T, _D)


# speedup vs baseline: 20.9466x; 1.2791x over previous
import functools
import math

import jax
import jax.numpy as jnp
from jax import lax
from jax.experimental import pallas as pl
from jax.experimental.pallas import tpu as pltpu

_B, _T, _D, _H = 1, 2048, 1024, 16
_Dh = _D // _H
_MAX_LAG = 32
_C_TOPK = 1


def _lag_candidates(t, max_lag):
    if t <= 1:
        return []
    if max_lag is None or max_lag >= t - 1:
        return list(range(1, t))
    step = max(1, (t - 1) // max_lag)
    cand = list(range(1, t, step))[:max_lag]
    if cand and cand[-1] != t - 1:
        cand[-1] = t - 1
    return cand


_LAGS = _lag_candidates(_T, _MAX_LAG)
_L = len(_LAGS)
_K_TOP = max(1, int(_C_TOPK * math.ceil(math.log(max(_T, 2), 2))))
_K_EFF = min(_K_TOP, _L)


def _tdot(a, b):
    return lax.dot_general(a, b, (((0,), (0,)), ((), ())),
                           preferred_element_type=jnp.float32)


def _softmax_rows(z):
    z = z - jnp.max(z, axis=-1, keepdims=True)
    e = jnp.exp(z)
    return e / jnp.sum(e, axis=-1, keepdims=True)


def _pair_body(qp, kp, vp, tau, lam, beta, tau_lag, qext_ref, covs_ref,
               yext_ref):
    W = 2 * _Dh
    qn = qp / jnp.sqrt(jnp.maximum(jnp.sum(qp * qp, axis=0, keepdims=True),
                                   1e-8))
    kn = kp / jnp.sqrt(jnp.maximum(jnp.sum(kp * kp, axis=0, keepdims=True),
                                   1e-8))

    qext_ref[0:_T, :] = qn
    qext_ref[_T:2 * _T, :] = qn

    cov0p = _tdot(kn, qn)

    eye = (lax.broadcasted_iota(jnp.int32, (_Dh, _Dh), 0) ==
           lax.broadcasted_iota(jnp.int32, (_Dh, _Dh), 1))
    scores = [[], []]
    for l, lag in enumerate(_LAGS):
        qw = qext_ref[lag:lag + _T, :]
        covp = _tdot(kn, qw)
        covs_ref[l * W:(l + 1) * W, :] = covp
        a = jnp.abs(covp)
        for s in range(2):
            ab = a[s * _Dh:(s + 1) * _Dh, s * _Dh:(s + 1) * _Dh]
            dsum = jnp.sum(jnp.where(eye, ab, 0.0))
            tsum = jnp.sum(ab)
            scores[s].append(
                (lam * dsum + (1.0 - lam) * (tsum - dsum)).reshape(1, 1))

    iota_l = lax.broadcasted_iota(jnp.int32, (1, _L), 1)
    outs = []
    for s in range(2):
        sl = slice(s * _Dh, (s + 1) * _Dh)
        v = vp[:, sl]
        cur = jnp.concatenate(scores[s], axis=1)
        sel_idx, sel_sc = [], []
        for _ in range(_K_EFF):
            m = jnp.max(cur)
            idx = jnp.min(jnp.where(cur == m, iota_l, _L))
            sel_idx.append(idx)
            sel_sc.append(m)
            cur = jnp.where(iota_l == idx, -jnp.inf, cur)

        mmax = functools.reduce(jnp.maximum, sel_sc)
        es = [jnp.exp((sc - mmax) / tau_lag) for sc in sel_sc]
        denom = functools.reduce(lax.add, es)

        att0 = _softmax_rows(cov0p[sl, sl] / tau)
        inst = jnp.dot(v, att0, preferred_element_type=jnp.float32)
        acc = (1.0 - beta) * inst
        for kk in range(_K_EFF):
            idx = sel_idx[kk]
            covk = covs_ref[pl.ds(idx * W + s * _Dh, _Dh), sl]
            attk = _softmax_rows(covk / tau)
            wk = beta * es[kk] / denom
            y = jnp.dot(v, attk * wk, preferred_element_type=jnp.float32)
            yext_ref[0:_T, :] = y
            yext_ref[_T:2 * _T, :] = y
            lagk = functools.reduce(lax.add, [
                jnp.where(idx == l, lag, 0) for l, lag in enumerate(_LAGS)])
            acc = acc + yext_ref[pl.ds(_T - lagk, _T), :]
        outs.append(acc)
    return jnp.concatenate(outs, axis=1)


def _fused_kernel(scal_ref, x_ref, wq_ref, wk_ref, wv_ref, bq_ref, bk_ref,
                  bv_ref, wo_ref, bo_ref, o_ref, qext_ref, covs_ref, yext_ref):
    g = pl.program_id(0)
    tau = scal_ref[0, 0]
    lam = scal_ref[0, 1]
    beta = scal_ref[0, 2]
    tau_lag = scal_ref[0, 3]

    xv = x_ref[:]
    qp = jnp.dot(xv, wq_ref[:], preferred_element_type=jnp.float32) + bq_ref[:]
    kp = jnp.dot(xv, wk_ref[:], preferred_element_type=jnp.float32) + bk_ref[:]
    vp = jnp.dot(xv, wv_ref[:], preferred_element_type=jnp.float32) + bv_ref[:]

    pair = _pair_body(qp, kp, vp, tau, lam, beta, tau_lag, qext_ref,
                      covs_ref, yext_ref)
    contrib = jnp.dot(pair, wo_ref[:], preferred_element_type=jnp.float32)

    @pl.when(g == 0)
    def _():
        o_ref[:] = contrib + bo_ref[:]

    @pl.when(g > 0)
    def _():
        o_ref[:] = o_ref[:] + contrib


def kernel(x, Wq, bq, Wk, bk, Wv, bv, Wo, bo, log_tau, lambda_auto, beta_lag,
           log_tau_lag):
    x2 = x.reshape(_T, _D)
    tau = jnp.maximum(jnp.exp(log_tau[0]), 1e-4)
    tau_lag = jnp.maximum(jnp.exp(log_tau_lag[0]), 1e-4)
    lam = jnp.clip(lambda_auto, 0.0, 1.0)
    beta = jnp.clip(beta_lag, 0.0, 1.0)
    scal = jnp.stack([tau, lam, beta, tau_lag]).reshape(1, 4)

    npair = _H // 2
    cw = 2 * _Dh
    out = pl.pallas_call(
        _fused_kernel,
        grid=(npair,),
        in_specs=[
            pl.BlockSpec(memory_space=pltpu.SMEM),
            pl.BlockSpec((_T, _D), lambda g: (0, 0)),
            pl.BlockSpec((_D, cw), lambda g: (0, g)),
            pl.BlockSpec((_D, cw), lambda g: (0, g)),
            pl.BlockSpec((_D, cw), lambda g: (0, g)),
            pl.BlockSpec((1, cw), lambda g: (0, g)),
            pl.BlockSpec((1, cw), lambda g: (0, g)),
            pl.BlockSpec((1, cw), lambda g: (0, g)),
            pl.BlockSpec((cw, _D), lambda g: (g, 0)),
            pl.BlockSpec((1, _D), lambda g: (0, 0)),
        ],
        out_specs=pl.BlockSpec((_T, _D), lambda g: (0, 0)),
        out_shape=jax.ShapeDtypeStruct((_T, _D), jnp.float32),
        scratch_shapes=[
            pltpu.VMEM((2 * _T, 2 * _Dh), jnp.float32),
            pltpu.VMEM((_L * 2 * _Dh, 2 * _Dh), jnp.float32),
            pltpu.VMEM((2 * _T, _Dh), jnp.float32),
        ],
    )(scal, x2, Wq.T, Wk.T, Wv.T, bq.reshape(1, _D), bk.reshape(1, _D),
      bv.reshape(1, _D), Wo.T, bo.reshape(1, _D))
    return out.reshape(_B, _T, _D)
